# Initial kernel scaffold; baseline (speedup 1.0000x reference)
import jax
import jax.numpy as jnp
from jax.experimental import pallas as pl


def kernel(x, W1, b1, W2, b2):
    raise NotImplementedError("write your pallas kernel here")


# speedup vs baseline: 7.4647x; 7.4647x over previous
import functools

import jax
import jax.numpy as jnp
from jax.experimental import pallas as pl

_K = 20
_CHUNK = 512


def _leaky(h):
    return jnp.where(h >= 0, h, 0.2 * h)


def _edgeconv_kernel(x_ref, amb_ref, bt_ref, w2t_ref, b1_ref, b2_ref, out_ref):
    n = x_ref.shape[2]
    chunk = out_ref.shape[2]
    c0 = pl.program_id(1) * chunk

    xf = x_ref[0]
    xx = jnp.sum(xf * xf, axis=0, keepdims=True)
    xc = x_ref[0, :, pl.ds(c0, chunk)].T

    mm = jnp.dot(xc, xf, preferred_element_type=jnp.float32)
    xx_r = jnp.sum(xc * xc, axis=1, keepdims=True)
    neg = (2.0 * mm - xx_r) - xx

    v_all = jnp.dot(xf.T, bt_ref[...], preferred_element_type=jnp.float32)
    u = jnp.dot(xc, amb_ref[...], preferred_element_type=jnp.float32) + b1_ref[0:1, :]

    iota = jax.lax.broadcasted_iota(jnp.int32, (chunk, n), 1)
    w2t = w2t_ref[...]
    b2 = b2_ref[0:1, :]

    def body(_, carry):
        negd, acc = carry
        m = jnp.max(negd, axis=1, keepdims=True)
        idx = jnp.min(jnp.where(negd == m, iota, n), axis=1, keepdims=True)
        onehot = (iota == idx)
        nbv = jnp.dot(onehot.astype(jnp.float32), v_all,
                      preferred_element_type=jnp.float32)
        h = _leaky(u + nbv)
        h = _leaky(jnp.dot(h, w2t, preferred_element_type=jnp.float32) + b2)
        acc = jnp.maximum(acc, h)
        negd = jnp.where(onehot, -jnp.inf, negd)
        return negd, acc

    acc0 = jnp.full((chunk, 64), -jnp.inf, dtype=jnp.float32)
    _, acc = jax.lax.fori_loop(0, _K, body, (neg, acc0))
    out_ref[0] = acc.T


@jax.jit
def kernel(x, W1, b1, W2, b2):
    B, C, N = x.shape
    xp = jnp.pad(x, ((0, 0), (0, 8 - C), (0, 0)))
    A = W1[:, :C]
    Bm = W1[:, C:]
    amb = jnp.pad((A - Bm).T, ((0, 8 - C), (0, 0)))
    bt = jnp.pad(Bm.T, ((0, 8 - C), (0, 0)))
    b1b = jnp.broadcast_to(b1[None, :], (8, 64))
    b2b = jnp.broadcast_to(b2[None, :], (8, 64))

    grid = (B, N // _CHUNK)
    out = pl.pallas_call(
        _edgeconv_kernel,
        grid=grid,
        in_specs=[
            pl.BlockSpec((1, 8, N), lambda b, c: (b, 0, 0)),
            pl.BlockSpec((8, 64), lambda b, c: (0, 0)),
            pl.BlockSpec((8, 64), lambda b, c: (0, 0)),
            pl.BlockSpec((64, 64), lambda b, c: (0, 0)),
            pl.BlockSpec((8, 64), lambda b, c: (0, 0)),
            pl.BlockSpec((8, 64), lambda b, c: (0, 0)),
        ],
        out_specs=pl.BlockSpec((1, 64, _CHUNK), lambda b, c: (b, 0, c)),
        out_shape=jax.ShapeDtypeStruct((B, 64, N), jnp.float32),
    )(xp, amb, bt, W2.T, b1b, b2b)
    return out


# speedup vs baseline: 10.4333x; 1.3977x over previous
import jax
import jax.numpy as jnp
from jax.experimental import pallas as pl
from jax.experimental.pallas import tpu as pltpu

_K = 20
_CHUNK = 512


def _leaky(h):
    return jnp.where(h >= 0, h, 0.2 * h)


def _edgeconv_kernel(x_ref, amb_ref, bt_ref, w2t_ref, b1_ref, b2_ref,
                     out_ref, neg_ref):
    n = x_ref.shape[2]
    chunk = out_ref.shape[2]
    c0 = pl.program_id(1) * chunk

    xf = x_ref[0]
    xx = jnp.sum(xf * xf, axis=0, keepdims=True)
    xc = x_ref[0, :, pl.ds(c0, chunk)].T

    mm = jnp.dot(xc, xf, preferred_element_type=jnp.float32)
    xx_r = jnp.sum(xc * xc, axis=1, keepdims=True)
    neg_ref[...] = (2.0 * mm - xx_r) - xx

    v_all = jnp.dot(xf.T, bt_ref[...], preferred_element_type=jnp.float32)
    u = jnp.dot(xc, amb_ref[...], preferred_element_type=jnp.float32) + b1_ref[0:1, :]

    iota = jax.lax.broadcasted_iota(jnp.int32, (chunk, n), 1)
    w2t = w2t_ref[...]
    b2 = b2_ref[0:1, :]

    def body(_, acc):
        d = neg_ref[...]
        m = jnp.max(d, axis=1, keepdims=True)
        idx = jnp.min(jnp.where(d == m, iota, n), axis=1, keepdims=True)
        onehot = iota == idx
        nbv = jnp.dot(onehot.astype(jnp.float32), v_all,
                      preferred_element_type=jnp.float32)
        h = _leaky(u + nbv)
        h = _leaky(jnp.dot(h, w2t, preferred_element_type=jnp.float32) + b2)
        neg_ref[...] = jnp.where(onehot, -jnp.inf, d)
        return jnp.maximum(acc, h)

    acc0 = jnp.full((chunk, 64), -jnp.inf, dtype=jnp.float32)
    acc = jax.lax.fori_loop(0, _K, body, acc0)
    out_ref[0] = acc.T


@jax.jit
def kernel(x, W1, b1, W2, b2):
    B, C, N = x.shape
    xp = jnp.pad(x, ((0, 0), (0, 8 - C), (0, 0)))
    A = W1[:, :C]
    Bm = W1[:, C:]
    amb = jnp.pad((A - Bm).T, ((0, 8 - C), (0, 0)))
    bt = jnp.pad(Bm.T, ((0, 8 - C), (0, 0)))
    b1b = jnp.broadcast_to(b1[None, :], (8, 64))
    b2b = jnp.broadcast_to(b2[None, :], (8, 64))

    grid = (B, N // _CHUNK)
    out = pl.pallas_call(
        _edgeconv_kernel,
        grid=grid,
        in_specs=[
            pl.BlockSpec((1, 8, N), lambda b, c: (b, 0, 0)),
            pl.BlockSpec((8, 64), lambda b, c: (0, 0)),
            pl.BlockSpec((8, 64), lambda b, c: (0, 0)),
            pl.BlockSpec((64, 64), lambda b, c: (0, 0)),
            pl.BlockSpec((8, 64), lambda b, c: (0, 0)),
            pl.BlockSpec((8, 64), lambda b, c: (0, 0)),
        ],
        out_specs=pl.BlockSpec((1, 64, _CHUNK), lambda b, c: (b, 0, c)),
        out_shape=jax.ShapeDtypeStruct((B, 64, N), jnp.float32),
        scratch_shapes=[pltpu.VMEM((_CHUNK, N), jnp.float32)],
    )(xp, amb, bt, W2.T, b1b, b2b)
    return out
